# baseline (speedup 1.0000x reference)
import functools

import jax
import jax.numpy as jnp
from jax import lax
from jax.experimental import pallas as pl
from jax.experimental.pallas import tpu as pltpu
from jax.experimental.pallas import tpu_sc as plsc

_N = 100000
_E = 6400000
_M = 1000
_KE = 27.211386245988 * 0.52917721067
_CUTOFF = 10.0
_SHIFT = 1.0 / _CUTOFF
_SCALE = 0.5 * _KE

_NW = 32
_EPT = _E // _NW
_C = 2000
_NCHUNK = _EPT // _C
_NPAIR = _NCHUNK // 2
_GROUPS = _C // 16
_MPAD = 1024

_TCB = 128000


def _pot_body(x_ref, y_ref, z_ref, o_ref):
    x = x_ref[...].astype(jnp.float32)
    y = y_ref[...].astype(jnp.float32)
    z = z_ref[...].astype(jnp.float32)
    r2 = x * x + y * y + z * z
    u = lax.rsqrt(r2)
    s2 = jnp.float32(_SHIFT * _SHIFT)
    pot = u * (1.0 + s2 * r2) - jnp.float32(2.0 * _SHIFT)
    pot = pot * jnp.float32(_SCALE)
    o_ref[...] = jnp.where(r2 <= jnp.float32(_CUTOFF * _CUTOFF), pot, 0.0)


_pot_call = pl.pallas_call(
    _pot_body,
    grid=(_E // _TCB,),
    in_specs=[pl.BlockSpec((_TCB,), lambda i: (i,))] * 3,
    out_specs=pl.BlockSpec((_TCB,), lambda i: (i,)),
    out_shape=jax.ShapeDtypeStruct((_E,), jnp.float32),
)


def _body(tbl_hbm, idxi_hbm, idxj_hbm, pot_hbm, out_hbm,
          tbl_v, idxi_v, idxj_v, pot_v, acc_v, tmp_v, shared, sem_a, sem_b):
    c = lax.axis_index("c")
    s = lax.axis_index("s")
    wid = s * 2 + c
    base_e = wid * _EPT

    def _fire(ci, half, sem):
        eb = base_e + ci * _C
        o = half * _C
        pltpu.async_copy(idxi_hbm.at[pl.ds(eb, _C)], idxi_v.at[pl.ds(o, _C)], sem)
        pltpu.async_copy(idxj_hbm.at[pl.ds(eb, _C)], idxj_v.at[pl.ds(o, _C)], sem)
        pltpu.async_copy(pot_hbm.at[pl.ds(eb, _C)], pot_v.at[pl.ds(o, _C)], sem)

    def _drain(ci, half, sem):
        eb = base_e + ci * _C
        o = half * _C
        pltpu.make_async_copy(idxi_hbm.at[pl.ds(eb, _C)], idxi_v.at[pl.ds(o, _C)], sem).wait()
        pltpu.make_async_copy(idxj_hbm.at[pl.ds(eb, _C)], idxj_v.at[pl.ds(o, _C)], sem).wait()
        pltpu.make_async_copy(pot_hbm.at[pl.ds(eb, _C)], pot_v.at[pl.ds(o, _C)], sem).wait()

    _fire(0, 0, sem_a)

    pltpu.sync_copy(tbl_hbm, tbl_v)

    zero16 = jnp.zeros((16,), jnp.float32)

    @plsc.parallel_loop(0, (16 * _MPAD) // 16, unroll=8)
    def _zero(i):
        acc_v[pl.ds(i * 16, 16)] = zero16

    lanes = lax.iota(jnp.int32, 16)
    lane_off = lanes * _MPAD
    himask = jnp.full((16,), -65536, jnp.int32)
    lomask = jnp.full((16,), 65535, jnp.int32)

    def _compute(half):
        o = half * _C

        @plsc.parallel_loop(0, _GROUPS, unroll=5)
        def _group(k):
            b = o + k * 16
            ii = idxi_v[pl.ds(b, 16)]
            jj = idxj_v[pl.ds(b, 16)]
            wi = plsc.load_gather(tbl_v, [ii])
            wj = plsc.load_gather(tbl_v, [jj])
            qi = plsc.bitcast(wi & himask, jnp.float32)
            qj = plsc.bitcast(wj & himask, jnp.float32)
            mi = wi & lomask
            val = qi * qj * pot_v[pl.ds(b, 16)]
            plsc.addupdate_scatter(acc_v, [mi + lane_off], val)

    def _pair(p, carry):
        ci0 = 2 * p
        _fire(ci0 + 1, 1, sem_b)
        _drain(ci0, 0, sem_a)
        _compute(0)

        @pl.when(p < _NPAIR - 1)
        def _():
            _fire(ci0 + 2, 0, sem_a)
        _drain(ci0 + 1, 1, sem_b)
        _compute(1)
        return carry
    lax.fori_loop(0, _NPAIR, _pair, 0)

    def _rrow(r, carry):
        def _radd(i, inner):
            acc_v[pl.ds(i * 16, 16)] = (
                acc_v[pl.ds(i * 16, 16)] + acc_v[pl.ds(r * _MPAD + i * 16, 16)])
            return inner
        lax.fori_loop(0, _MPAD // 16, _radd, 0)
        return carry
    lax.fori_loop(1, 16, _rrow, 0)

    pltpu.sync_copy(acc_v.at[pl.ds(0, _MPAD)], shared.at[s])
    plsc.subcore_barrier()

    @pl.when(s == 0)
    def _combine():
        def _srow(r, carry):
            pltpu.sync_copy(shared.at[r], tmp_v)

            def _sadd(i, inner):
                acc_v[pl.ds(i * 16, 16)] = (
                    acc_v[pl.ds(i * 16, 16)] + tmp_v[pl.ds(i * 16, 16)])
                return inner
            lax.fori_loop(0, _MPAD // 16, _sadd, 0)
            return carry
        lax.fori_loop(1, 16, _srow, 0)
        pltpu.sync_copy(acc_v.at[pl.ds(0, _MPAD)], out_hbm.at[pl.ds(c * _MPAD, _MPAD)])


_sc_call = functools.partial(
    pl.kernel,
    out_type=jax.ShapeDtypeStruct((2 * _MPAD,), jnp.float32),
    mesh=plsc.VectorSubcoreMesh(core_axis_name="c", subcore_axis_name="s"),
    scratch_types=[
        pltpu.VMEM((_N,), jnp.int32),
        pltpu.VMEM((2 * _C,), jnp.int32),
        pltpu.VMEM((2 * _C,), jnp.int32),
        pltpu.VMEM((2 * _C,), jnp.float32),
        pltpu.VMEM((16 * _MPAD,), jnp.float32),
        pltpu.VMEM((_MPAD,), jnp.float32),
        pltpu.VMEM_SHARED((16, _MPAD), jnp.float32),
        pltpu.SemaphoreType.DMA,
        pltpu.SemaphoreType.DMA,
    ],
    compiler_params=pltpu.CompilerParams(needs_layout_passes=False),
)(_body)


def kernel(partial_charges, Rij_lr, idx_i_lr, idx_j_lr, idx_m):
    q = partial_charges.reshape(-1)
    qbits = lax.bitcast_convert_type(
        q.astype(jnp.bfloat16), jnp.uint16).astype(jnp.uint32) << 16
    tbl = lax.bitcast_convert_type(
        qbits | idx_m.astype(jnp.uint32), jnp.int32)
    ii = idx_i_lr.astype(jnp.int32)
    jj = idx_j_lr.astype(jnp.int32)
    pot = _pot_call(Rij_lr[:, 0].astype(jnp.bfloat16),
                    Rij_lr[:, 1].astype(jnp.bfloat16),
                    Rij_lr[:, 2].astype(jnp.bfloat16))
    out2 = _sc_call(tbl, ii, jj, pot).reshape(2, _MPAD)
    return out2[0, :_M] + out2[1, :_M]


# speedup vs baseline: 3.3234x; 3.3234x over previous
import functools

import jax
import jax.numpy as jnp
from jax import lax
from jax.experimental import pallas as pl
from jax.experimental.pallas import tpu as pltpu
from jax.experimental.pallas import tpu_sc as plsc

_N = 100000
_E = 6400000
_M = 1000
_KE = 27.211386245988 * 0.52917721067
_CUTOFF = 10.0
_SHIFT = 1.0 / _CUTOFF
_SCALE = 0.5 * _KE

_NW = 32
_NPH = 2
_EH = _E // _NPH
_EPT = _EH // _NW
_C = 2000
_NCHUNK = _EPT // _C
_NPAIR = _NCHUNK // 2
_GROUPS = _C // 16
_MPAD = 1024

_TCB = 128000


def _pot_body(x_ref, y_ref, z_ref, o_ref):
    x = x_ref[...]
    y = y_ref[...]
    z = z_ref[...]
    r2 = x * x + y * y + z * z
    u = lax.rsqrt(r2)
    s2 = jnp.float32(_SHIFT * _SHIFT)
    pot = u * (1.0 + s2 * r2) - jnp.float32(2.0 * _SHIFT)
    pot = pot * jnp.float32(_SCALE)
    o_ref[...] = jnp.where(r2 <= jnp.float32(_CUTOFF * _CUTOFF), pot, 0.0)


_pot_call = pl.pallas_call(
    _pot_body,
    grid=(_EH // _TCB,),
    in_specs=[pl.BlockSpec((_TCB,), lambda i: (i,))] * 3,
    out_specs=pl.BlockSpec((_TCB,), lambda i: (i,)),
    out_shape=jax.ShapeDtypeStruct((_EH,), jnp.float32),
)


def _body(phase, tbl_hbm, idxi_hbm, idxj_hbm, pot_hbm, out_hbm,
          tbl_v, idxi_v, idxj_v, pot_v, acc_v, tmp_v, shared, sem_a, sem_b):
    c = lax.axis_index("c")
    s = lax.axis_index("s")
    wid = s * 2 + c
    base_p = wid * _EPT
    base_e = phase * _EH + base_p

    def _fire(ci, half, sem):
        eb = base_e + ci * _C
        pb = base_p + ci * _C
        o = half * _C
        pltpu.async_copy(idxi_hbm.at[pl.ds(eb, _C)], idxi_v.at[pl.ds(o, _C)], sem)
        pltpu.async_copy(idxj_hbm.at[pl.ds(eb, _C)], idxj_v.at[pl.ds(o, _C)], sem)
        pltpu.async_copy(pot_hbm.at[pl.ds(pb, _C)], pot_v.at[pl.ds(o, _C)], sem)

    def _drain(ci, half, sem):
        eb = base_e + ci * _C
        pb = base_p + ci * _C
        o = half * _C
        pltpu.make_async_copy(idxi_hbm.at[pl.ds(eb, _C)], idxi_v.at[pl.ds(o, _C)], sem).wait()
        pltpu.make_async_copy(idxj_hbm.at[pl.ds(eb, _C)], idxj_v.at[pl.ds(o, _C)], sem).wait()
        pltpu.make_async_copy(pot_hbm.at[pl.ds(pb, _C)], pot_v.at[pl.ds(o, _C)], sem).wait()

    _fire(0, 0, sem_a)

    pltpu.sync_copy(tbl_hbm, tbl_v)

    zero16 = jnp.zeros((16,), jnp.float32)

    @plsc.parallel_loop(0, (16 * _MPAD) // 16, unroll=8)
    def _zero(i):
        acc_v[pl.ds(i * 16, 16)] = zero16

    lanes = lax.iota(jnp.int32, 16)
    lane_off = lanes * _MPAD
    himask = jnp.full((16,), -65536, jnp.int32)
    lomask = jnp.full((16,), 65535, jnp.int32)

    def _compute(half):
        o = half * _C

        @plsc.parallel_loop(0, _GROUPS, unroll=5)
        def _group(k):
            b = o + k * 16
            ii = idxi_v[pl.ds(b, 16)]
            jj = idxj_v[pl.ds(b, 16)]
            wi = plsc.load_gather(tbl_v, [ii])
            wj = plsc.load_gather(tbl_v, [jj])
            qi = plsc.bitcast(wi & himask, jnp.float32)
            qj = plsc.bitcast(wj & himask, jnp.float32)
            mi = wi & lomask
            val = qi * qj * pot_v[pl.ds(b, 16)]
            plsc.addupdate_scatter(acc_v, [mi + lane_off], val)

    def _pair(p, carry):
        ci0 = 2 * p
        _fire(ci0 + 1, 1, sem_b)
        _drain(ci0, 0, sem_a)
        _compute(0)

        @pl.when(p < _NPAIR - 1)
        def _():
            _fire(ci0 + 2, 0, sem_a)
        _drain(ci0 + 1, 1, sem_b)
        _compute(1)
        return carry
    lax.fori_loop(0, _NPAIR, _pair, 0)

    def _rrow(r, carry):
        def _radd(i, inner):
            acc_v[pl.ds(i * 16, 16)] = (
                acc_v[pl.ds(i * 16, 16)] + acc_v[pl.ds(r * _MPAD + i * 16, 16)])
            return inner
        lax.fori_loop(0, _MPAD // 16, _radd, 0)
        return carry
    lax.fori_loop(1, 16, _rrow, 0)

    pltpu.sync_copy(acc_v.at[pl.ds(0, _MPAD)], shared.at[s])
    plsc.subcore_barrier()

    @pl.when(s == 0)
    def _combine():
        def _srow(r, carry):
            pltpu.sync_copy(shared.at[r], tmp_v)

            def _sadd(i, inner):
                acc_v[pl.ds(i * 16, 16)] = (
                    acc_v[pl.ds(i * 16, 16)] + tmp_v[pl.ds(i * 16, 16)])
                return inner
            lax.fori_loop(0, _MPAD // 16, _sadd, 0)
            return carry
        lax.fori_loop(1, 16, _srow, 0)
        pltpu.sync_copy(acc_v.at[pl.ds(0, _MPAD)], out_hbm.at[pl.ds(c * _MPAD, _MPAD)])


def _make_sc_call(phase):
    return functools.partial(
        pl.kernel,
        out_type=jax.ShapeDtypeStruct((2 * _MPAD,), jnp.float32),
        mesh=plsc.VectorSubcoreMesh(core_axis_name="c", subcore_axis_name="s"),
        scratch_types=[
            pltpu.VMEM((_N,), jnp.int32),
            pltpu.VMEM((2 * _C,), jnp.int32),
            pltpu.VMEM((2 * _C,), jnp.int32),
            pltpu.VMEM((2 * _C,), jnp.float32),
            pltpu.VMEM((16 * _MPAD,), jnp.float32),
            pltpu.VMEM((_MPAD,), jnp.float32),
            pltpu.VMEM_SHARED((16, _MPAD), jnp.float32),
            pltpu.SemaphoreType.DMA,
            pltpu.SemaphoreType.DMA,
        ],
        compiler_params=pltpu.CompilerParams(needs_layout_passes=False),
    )(functools.partial(_body, phase))


_sc_call_0 = _make_sc_call(0)
_sc_call_1 = _make_sc_call(1)


def kernel(partial_charges, Rij_lr, idx_i_lr, idx_j_lr, idx_m):
    q = partial_charges.reshape(-1)
    qbits = lax.bitcast_convert_type(
        q.astype(jnp.bfloat16), jnp.uint16).astype(jnp.uint32) << 16
    tbl = lax.bitcast_convert_type(
        qbits | idx_m.astype(jnp.uint32), jnp.int32)
    ii = idx_i_lr.astype(jnp.int32)
    jj = idx_j_lr.astype(jnp.int32)
    pot_a = _pot_call(Rij_lr[:_EH, 0], Rij_lr[:_EH, 1], Rij_lr[:_EH, 2])
    out_a = _sc_call_0(tbl, ii, jj, pot_a).reshape(2, _MPAD)
    pot_b = _pot_call(Rij_lr[_EH:, 0], Rij_lr[_EH:, 1], Rij_lr[_EH:, 2])
    out_b = _sc_call_1(tbl, ii, jj, pot_b).reshape(2, _MPAD)
    out2 = out_a + out_b
    return out2[0, :_M] + out2[1, :_M]


# speedup vs baseline: 3.5273x; 1.0613x over previous
import functools

import jax
import jax.numpy as jnp
from jax import lax
from jax.experimental import pallas as pl
from jax.experimental.pallas import tpu as pltpu
from jax.experimental.pallas import tpu_sc as plsc

_N = 100000
_E = 6400000
_M = 1000
_KE = 27.211386245988 * 0.52917721067
_CUTOFF = 10.0
_SHIFT = 1.0 / _CUTOFF
_SCALE = 0.5 * _KE

_NW = 32
_EPT = _E // _NW
_C = 2000
_NCHUNK = _EPT // _C
_NPAIR = _NCHUNK // 2
_GROUPS = _C // 16
_MPAD = 1024

_TCB = 128000


def _pot_body(x_ref, y_ref, z_ref, o_ref):
    x = x_ref[...]
    y = y_ref[...]
    z = z_ref[...]
    r2 = x * x + y * y + z * z
    u = lax.rsqrt(r2)
    s2 = jnp.float32(_SHIFT * _SHIFT)
    pot = u * (1.0 + s2 * r2) - jnp.float32(2.0 * _SHIFT)
    pot = pot * jnp.float32(_SCALE)
    o_ref[...] = jnp.where(r2 <= jnp.float32(_CUTOFF * _CUTOFF), pot, 0.0)


_pot_call = pl.pallas_call(
    _pot_body,
    grid=(_E // _TCB,),
    in_specs=[pl.BlockSpec((_TCB,), lambda i: (i,))] * 3,
    out_specs=pl.BlockSpec((_TCB,), lambda i: (i,)),
    out_shape=jax.ShapeDtypeStruct((_E,), jnp.float32),
)


def _body(tbl_hbm, idxi_hbm, idxj_hbm, pot_hbm, out_hbm,
          tbl_v, idxi_v, idxj_v, pot_v, acc_v, tmp_v, shared, sem_a, sem_b):
    c = lax.axis_index("c")
    s = lax.axis_index("s")
    wid = s * 2 + c
    base_e = wid * _EPT

    def _fire(ci, half, sem):
        eb = base_e + ci * _C
        o = half * _C
        pltpu.async_copy(idxi_hbm.at[pl.ds(eb, _C)], idxi_v.at[pl.ds(o, _C)], sem)
        pltpu.async_copy(idxj_hbm.at[pl.ds(eb, _C)], idxj_v.at[pl.ds(o, _C)], sem)
        pltpu.async_copy(pot_hbm.at[pl.ds(eb, _C)], pot_v.at[pl.ds(o, _C)], sem)

    def _drain(ci, half, sem):
        eb = base_e + ci * _C
        o = half * _C
        pltpu.make_async_copy(idxi_hbm.at[pl.ds(eb, _C)], idxi_v.at[pl.ds(o, _C)], sem).wait()
        pltpu.make_async_copy(idxj_hbm.at[pl.ds(eb, _C)], idxj_v.at[pl.ds(o, _C)], sem).wait()
        pltpu.make_async_copy(pot_hbm.at[pl.ds(eb, _C)], pot_v.at[pl.ds(o, _C)], sem).wait()

    _fire(0, 0, sem_a)

    pltpu.sync_copy(tbl_hbm, tbl_v)

    zero16 = jnp.zeros((16,), jnp.float32)

    @plsc.parallel_loop(0, (16 * _MPAD) // 16, unroll=8)
    def _zero(i):
        acc_v[pl.ds(i * 16, 16)] = zero16

    lanes = lax.iota(jnp.int32, 16)
    lane_off = lanes * _MPAD
    himask = jnp.full((16,), -65536, jnp.int32)
    lomask = jnp.full((16,), 65535, jnp.int32)

    def _compute(half):
        o = half * _C

        @plsc.parallel_loop(0, _GROUPS, unroll=5)
        def _group(k):
            b = o + k * 16
            ii = idxi_v[pl.ds(b, 16)]
            jj = idxj_v[pl.ds(b, 16)]
            wi = plsc.load_gather(tbl_v, [ii])
            wj = plsc.load_gather(tbl_v, [jj])
            qi = plsc.bitcast(wi & himask, jnp.float32)
            qj = plsc.bitcast(wj & himask, jnp.float32)
            mi = wi & lomask
            val = qi * qj * pot_v[pl.ds(b, 16)]
            plsc.addupdate_scatter(acc_v, [mi + lane_off], val)

    def _pair(p, carry):
        ci0 = 2 * p
        _fire(ci0 + 1, 1, sem_b)
        _drain(ci0, 0, sem_a)
        _compute(0)

        @pl.when(p < _NPAIR - 1)
        def _():
            _fire(ci0 + 2, 0, sem_a)
        _drain(ci0 + 1, 1, sem_b)
        _compute(1)
        return carry
    lax.fori_loop(0, _NPAIR, _pair, 0)

    def _rrow(r, carry):
        def _radd(i, inner):
            acc_v[pl.ds(i * 16, 16)] = (
                acc_v[pl.ds(i * 16, 16)] + acc_v[pl.ds(r * _MPAD + i * 16, 16)])
            return inner
        lax.fori_loop(0, _MPAD // 16, _radd, 0)
        return carry
    lax.fori_loop(1, 16, _rrow, 0)

    pltpu.sync_copy(acc_v.at[pl.ds(0, _MPAD)], shared.at[s])
    plsc.subcore_barrier()

    @pl.when(s == 0)
    def _combine():
        def _srow(r, carry):
            pltpu.sync_copy(shared.at[r], tmp_v)

            def _sadd(i, inner):
                acc_v[pl.ds(i * 16, 16)] = (
                    acc_v[pl.ds(i * 16, 16)] + tmp_v[pl.ds(i * 16, 16)])
                return inner
            lax.fori_loop(0, _MPAD // 16, _sadd, 0)
            return carry
        lax.fori_loop(1, 16, _srow, 0)
        pltpu.sync_copy(acc_v.at[pl.ds(0, _MPAD)], out_hbm.at[pl.ds(c * _MPAD, _MPAD)])


_sc_call = functools.partial(
    pl.kernel,
    out_type=jax.ShapeDtypeStruct((2 * _MPAD,), jnp.float32),
    mesh=plsc.VectorSubcoreMesh(core_axis_name="c", subcore_axis_name="s"),
    scratch_types=[
        pltpu.VMEM((_N,), jnp.int32),
        pltpu.VMEM((2 * _C,), jnp.int32),
        pltpu.VMEM((2 * _C,), jnp.int32),
        pltpu.VMEM((2 * _C,), jnp.float32),
        pltpu.VMEM((16 * _MPAD,), jnp.float32),
        pltpu.VMEM((_MPAD,), jnp.float32),
        pltpu.VMEM_SHARED((16, _MPAD), jnp.float32),
        pltpu.SemaphoreType.DMA,
        pltpu.SemaphoreType.DMA,
    ],
    compiler_params=pltpu.CompilerParams(needs_layout_passes=False),
)(_body)


def kernel(partial_charges, Rij_lr, idx_i_lr, idx_j_lr, idx_m):
    q = partial_charges.reshape(-1)
    qbits = lax.bitcast_convert_type(
        q.astype(jnp.bfloat16), jnp.uint16).astype(jnp.uint32) << 16
    tbl = lax.bitcast_convert_type(
        qbits | idx_m.astype(jnp.uint32), jnp.int32)
    ii = idx_i_lr.astype(jnp.int32)
    jj = idx_j_lr.astype(jnp.int32)
    pot = _pot_call(Rij_lr[:, 0], Rij_lr[:, 1], Rij_lr[:, 2])
    out2 = _sc_call(tbl, ii, jj, pot).reshape(2, _MPAD)
    return out2[0, :_M] + out2[1, :_M]
